# baseline (speedup 1.0000x reference)
import functools

import jax
import jax.numpy as jnp
from jax import lax
from jax.experimental import pallas as pl
from jax.experimental.pallas import tpu as pltpu
from jax.experimental.pallas import tpu_sc as plsc

USER_NUM = 1000000
ITEM_NUM = 100000
DIM = 64
BATCH = 16384

NC = 2
NS = 16
NW = NC * NS
B_PER_W = BATCH // NW

U_ALIGNED = (USER_NUM // 128) * 128
U_CHUNK = 512
U_TAIL = USER_NUM - U_ALIGNED
I_ALIGNED = (ITEM_NUM // 128) * 128
I_CHUNK = 128
I_TAIL = ITEM_NUM - I_ALIGNED

_MESH = dict(core_axis_name="c", subcore_axis_name="s")
_PARAMS = pltpu.CompilerParams(needs_layout_passes=False)


def _sweep_body(uf_hbm, if_hbm, ut_hbm, it_hbm, tu_hbm, ti_hbm,
                ru_hbm, ri_hbm,
                idx_s, vals, buf, stg_v, stage, pos2,
                sem_c, sem_s):
    wid = lax.axis_index("s") * NC + lax.axis_index("c")
    lane = lax.iota(jnp.int32, 16)
    dummy_pos = BATCH + lane

    def run_phase(idx_hbm, tab_hbm, tail_hbm, rows_hbm,
                  chunk, aligned, tail_lanes, lo, hi, trips):
        pltpu.sync_copy(idx_hbm, idx_s)
        pltpu.sync_copy(tail_hbm, buf.at[:, pl.ds(2 * chunk, 128)])

        def pre_body(j, cnt):
            v = idx_s[j // 8, pl.ds((j % 8) * 16, 16)]
            b = j * 16 + lane
            m = (v >= lo) & (v < hi)
            packed = lax.shift_left(v - lo, 14) | b
            plsc.store_compressed(vals.at[pl.ds(cnt, 16)], packed, mask=m)
            return cnt + plsc.all_reduce_population_count(m)[0]

        cnt = lax.fori_loop(0, BATCH // 16, pre_body, 0)
        nv = (cnt + 15) // 16
        vals[pl.ds(cnt, 16)] = jnp.full((16,), 0x7FFFFFFF, jnp.int32)

        def drain_s():
            pltpu.make_async_copy(
                rows_hbm.at[pl.ds(0, 16)], stage.at[0], sem_s).wait()

        nsplit = 2 if chunk % 256 == 0 else 1
        half = chunk // nsplit

        def wait_c():
            for _ in range(nsplit):
                pltpu.make_async_copy(
                    tab_hbm.at[:, pl.ds(0, half)],
                    buf.at[:, pl.ds(0, half)], sem_c).wait()

        def issue_c(t):
            for s in range(nsplit):
                pltpu.async_copy(
                    tab_hbm.at[:, pl.ds(lo + t * chunk + s * half, half)],
                    buf.at[:, pl.ds((t & 1) * chunk + s * half, half)], sem_c)

        def scan_list(rel_base, size, col_off, carry0):
            def vec_body(j, carry):
                p = vals[pl.ds(j * 16, 16)]
                pv = lax.shift_right_logical(p, 14)
                m = (pv >= rel_base) & (pv < rel_base + size)
                plsc.store_compressed(stg_v.at[pl.ds(0, 16)], p, mask=m)
                n = plsc.all_reduce_population_count(m)[0]

                def hit(l, hc):
                    hh, pvec = hc
                    pp = stg_v[pl.ds(l, 16)][0]
                    col = lax.shift_right_logical(pp, 14) - rel_base + col_off
                    ib = pp & 16383
                    g = hh >> 4
                    slot = hh & 15
                    par = g & 1
                    pvec = jnp.where(lane == slot, ib, pvec)

                    colvec = jnp.broadcast_to(col, (16,))
                    for k in range(DIM // 16):
                        stage[par, slot, pl.ds(k * 16, 16)] = plsc.load_gather(
                            buf, [k * 16 + lane, colvec])

                    @pl.when(slot == 15)
                    def _():
                        @pl.when(g >= 1)
                        def _():
                            drain_s()

                        pos2[par] = pvec
                        pltpu.async_copy(
                            stage.at[par], rows_hbm.at[pos2.at[par]], sem_s)

                    pvec = jnp.where(slot == 15, dummy_pos, pvec)
                    return hh + 1, pvec

                return lax.fori_loop(0, n, hit, carry)

            return lax.fori_loop(0, nv, vec_body, carry0)

        issue_c(0)

        def chunk_body(t, carry):
            wait_c()

            @pl.when(t + 1 < trips)
            def _():
                issue_c(t + 1)

            return scan_list(t * chunk, chunk, (t & 1) * chunk, carry)

        carry = lax.fori_loop(0, trips, chunk_body, (0, dummy_pos))

        h, pvec = scan_list(aligned - lo, tail_lanes, 2 * chunk, carry)

        @pl.when((h & 15) != 0)
        def _():
            @pl.when(h >= 16)
            def _():
                drain_s()

            par = (h >> 4) & 1
            pos2[par] = pvec
            pltpu.async_copy(stage.at[par], rows_hbm.at[pos2.at[par]], sem_s)

        @pl.when(h > 0)
        def _():
            drain_s()

    u_lo = wid * 244 * 128
    run_phase(
        uf_hbm, ut_hbm, tu_hbm, ru_hbm, U_CHUNK, U_ALIGNED, U_TAIL,
        u_lo, jnp.where(wid == NW - 1, USER_NUM, u_lo + 244 * 128),
        jnp.where(wid == NW - 1, 62, 61))

    i_lo = (wid * 24 + jnp.minimum(wid, 13)) * 128
    i_nw = jnp.where(wid < 13, 25, 24)
    run_phase(
        if_hbm, it_hbm, ti_hbm, ri_hbm, I_CHUNK, I_ALIGNED, I_TAIL,
        i_lo, jnp.where(wid == NW - 1, ITEM_NUM, i_lo + i_nw * 128),
        i_nw)


def _combine_body(ru_hbm, ri_hbm, out_hbm, su, si, out_v):
    wid = lax.axis_index("s") * NC + lax.axis_index("c")
    lane = lax.iota(jnp.int32, 16)
    slab = B_PER_W // 8

    def q_body(q, _):
        off = wid * B_PER_W + q * slab
        pltpu.sync_copy(ru_hbm.at[pl.ds(off, slab)], su)
        pltpu.sync_copy(ri_hbm.at[pl.ds(off, slab)], si)

        def group_body(g, _):
            base = g * 16
            res = jnp.zeros((16,), jnp.float32)
            for j in range(16):
                r = base + j
                acc = jnp.zeros((16,), jnp.float32)
                for k in range(DIM // 16):
                    acc = acc + su[r, pl.ds(k * 16, 16)] * si[r, pl.ds(k * 16, 16)]
                res = jnp.where(lane == j, jnp.sum(acc), res)
            out_v[pl.ds(q * slab + base, 16)] = res
            return 0

        lax.fori_loop(0, slab // 16, group_body, 0)
        return 0

    lax.fori_loop(0, 8, q_body, 0)
    pltpu.sync_copy(out_v, out_hbm.at[pl.ds(wid * B_PER_W, B_PER_W)])


@jax.jit
def _run(uf2d, if2d, ut_t, it_t, tail_u, tail_i):
    rows_t = jax.ShapeDtypeStruct((BATCH + 16, 128), jnp.float32)
    sweep = functools.partial(
        pl.kernel,
        out_type=(rows_t, rows_t),
        mesh=plsc.VectorSubcoreMesh(**_MESH),
        compiler_params=_PARAMS,
        scratch_types=[
            pltpu.VMEM((BATCH // 128, 128), jnp.int32),
            pltpu.VMEM((BATCH + 16,), jnp.int32),
            pltpu.VMEM((DIM, 2 * U_CHUNK + 128), jnp.float32),
            pltpu.VMEM((48,), jnp.int32),
            pltpu.VMEM((2, 16, 128), jnp.float32),
            pltpu.VMEM((2, 16), jnp.int32),
            pltpu.SemaphoreType.DMA,
            pltpu.SemaphoreType.DMA,
        ],
    )(_sweep_body)
    rows_u, rows_i = sweep(uf2d, if2d, ut_t, it_t, tail_u, tail_i)

    comb = functools.partial(
        pl.kernel,
        out_type=jax.ShapeDtypeStruct((BATCH,), jnp.float32),
        mesh=plsc.VectorSubcoreMesh(**_MESH),
        compiler_params=_PARAMS,
        scratch_types=[
            pltpu.VMEM((B_PER_W // 8, 128), jnp.float32),
            pltpu.VMEM((B_PER_W // 8, 128), jnp.float32),
            pltpu.VMEM((B_PER_W,), jnp.float32),
        ],
    )(_combine_body)
    return comb(rows_u, rows_i)


def kernel(user_feature, item_feature, user_table, item_table):
    uf2d = user_feature.astype(jnp.int32).reshape(BATCH // 128, 128)
    if2d = item_feature.astype(jnp.int32).reshape(BATCH // 128, 128)
    ut_t = user_table.T
    it_t = item_table.T
    tail_u = jnp.pad(user_table[U_ALIGNED:].T, ((0, 0), (0, 128 - U_TAIL)))
    tail_i = jnp.pad(item_table[I_ALIGNED:].T, ((0, 0), (0, 128 - I_TAIL)))
    return _run(uf2d, if2d, ut_t, it_t, tail_u, tail_i)


# speedup vs baseline: 1.0268x; 1.0268x over previous
import functools

import jax
import jax.numpy as jnp
from jax import lax
from jax.experimental import pallas as pl
from jax.experimental.pallas import tpu as pltpu
from jax.experimental.pallas import tpu_sc as plsc

USER_NUM = 1000000
ITEM_NUM = 100000
DIM = 64
BATCH = 16384

NC = 2
NS = 16
NW = NC * NS
B_PER_W = BATCH // NW

U_ALIGNED = (USER_NUM // 128) * 128
U_CHUNK = 512
U_TAIL = USER_NUM - U_ALIGNED
I_CHUNK = 384
I_ALIGNED = 780 * 128
I_TAIL = ITEM_NUM - I_ALIGNED

_MESH = dict(core_axis_name="c", subcore_axis_name="s")
_PARAMS = pltpu.CompilerParams(needs_layout_passes=False)


def _sweep_body(uf_hbm, if_hbm, ut_hbm, it_hbm, tu_hbm, ti_hbm,
                ru_hbm, ri_hbm,
                idx_s, vals, buf, stg_v, stage, pos2,
                sem_c, sem_s):
    wid = lax.axis_index("s") * NC + lax.axis_index("c")
    lane = lax.iota(jnp.int32, 16)
    dummy_pos = BATCH + lane

    def run_phase(idx_hbm, tab_hbm, tail_hbm, rows_hbm,
                  chunk, aligned, tail_lanes, tail_w, lo, hi, trips):
        pltpu.sync_copy(idx_hbm, idx_s)
        pltpu.sync_copy(tail_hbm, buf.at[:, pl.ds(2 * chunk, tail_w)])

        def pre_body(j, cnt):
            v = idx_s[j // 8, pl.ds((j % 8) * 16, 16)]
            b = j * 16 + lane
            m = (v >= lo) & (v < hi)
            packed = lax.shift_left(v - lo, 14) | b
            plsc.store_compressed(vals.at[pl.ds(cnt, 16)], packed, mask=m)
            return cnt + plsc.all_reduce_population_count(m)[0]

        cnt = lax.fori_loop(0, BATCH // 16, pre_body, 0, unroll=4)
        nv = (cnt + 15) // 16
        vals[pl.ds(cnt, 16)] = jnp.full((16,), 0x7FFFFFFF, jnp.int32)

        def drain_s():
            pltpu.make_async_copy(
                rows_hbm.at[pl.ds(0, 16)], stage.at[0], sem_s).wait()

        nsplit = 2 if chunk % 256 == 0 else 1
        half = chunk // nsplit

        def wait_c():
            for _ in range(nsplit):
                pltpu.make_async_copy(
                    tab_hbm.at[:, pl.ds(0, half)],
                    buf.at[:, pl.ds(0, half)], sem_c).wait()

        def issue_c(t):
            for s in range(nsplit):
                pltpu.async_copy(
                    tab_hbm.at[:, pl.ds(lo + t * chunk + s * half, half)],
                    buf.at[:, pl.ds((t & 1) * chunk + s * half, half)], sem_c)

        def scan_list(rel_base, size, col_off, carry0):
            def vec_body(j, carry):
                p = vals[pl.ds(j * 16, 16)]
                pv = lax.shift_right_logical(p, 14)
                m = (pv >= rel_base) & (pv < rel_base + size)
                plsc.store_compressed(stg_v.at[pl.ds(0, 16)], p, mask=m)
                n = plsc.all_reduce_population_count(m)[0]

                def hit(l, hc):
                    hh, pvec = hc
                    pp = stg_v[pl.ds(l, 16)][0]
                    col = lax.shift_right_logical(pp, 14) - rel_base + col_off
                    ib = pp & 16383
                    g = hh >> 4
                    slot = hh & 15
                    par = g & 1
                    pvec = jnp.where(lane == slot, ib, pvec)

                    colvec = jnp.broadcast_to(col, (16,))
                    for k in range(DIM // 16):
                        stage[par, slot, pl.ds(k * 16, 16)] = plsc.load_gather(
                            buf, [k * 16 + lane, colvec])

                    @pl.when(slot == 15)
                    def _():
                        @pl.when(g >= 1)
                        def _():
                            drain_s()

                        pos2[par] = pvec
                        pltpu.async_copy(
                            stage.at[par], rows_hbm.at[pos2.at[par]], sem_s)

                    pvec = jnp.where(slot == 15, dummy_pos, pvec)
                    return hh + 1, pvec

                return lax.fori_loop(0, n, hit, carry)

            return lax.fori_loop(0, nv, vec_body, carry0)

        issue_c(0)

        def chunk_body(t, carry):
            wait_c()

            @pl.when(t + 1 < trips)
            def _():
                issue_c(t + 1)

            return scan_list(t * chunk, chunk, (t & 1) * chunk, carry)

        carry = lax.fori_loop(0, trips, chunk_body, (0, dummy_pos))

        h, pvec = scan_list(aligned - lo, tail_lanes, 2 * chunk, carry)

        @pl.when((h & 15) != 0)
        def _():
            @pl.when(h >= 16)
            def _():
                drain_s()

            par = (h >> 4) & 1
            pos2[par] = pvec
            pltpu.async_copy(stage.at[par], rows_hbm.at[pos2.at[par]], sem_s)

        @pl.when(h > 0)
        def _():
            drain_s()

    u_lo = wid * 244 * 128
    run_phase(
        uf_hbm, ut_hbm, tu_hbm, ru_hbm, U_CHUNK, U_ALIGNED, U_TAIL, 128,
        u_lo, jnp.where(wid == NW - 1, USER_NUM, u_lo + 244 * 128),
        jnp.where(wid == NW - 1, 62, 61))

    i_lo = (wid * 24 + 3 * jnp.minimum(wid, 4)) * 128
    i_nw = jnp.where(wid < 4, 27, 24)
    run_phase(
        if_hbm, it_hbm, ti_hbm, ri_hbm, I_CHUNK, I_ALIGNED, I_TAIL, 256,
        i_lo, jnp.where(wid == NW - 1, ITEM_NUM, i_lo + i_nw * 128),
        jnp.where(wid < 4, 9, 8))


def _combine_body(ru_hbm, ri_hbm, out_hbm, su, si, out_v):
    wid = lax.axis_index("s") * NC + lax.axis_index("c")
    lane = lax.iota(jnp.int32, 16)
    slab = B_PER_W // 8

    def q_body(q, _):
        off = wid * B_PER_W + q * slab
        pltpu.sync_copy(ru_hbm.at[pl.ds(off, slab)], su)
        pltpu.sync_copy(ri_hbm.at[pl.ds(off, slab)], si)

        def group_body(g, _):
            base = g * 16
            res = jnp.zeros((16,), jnp.float32)
            for j in range(16):
                r = base + j
                acc = jnp.zeros((16,), jnp.float32)
                for k in range(DIM // 16):
                    acc = acc + su[r, pl.ds(k * 16, 16)] * si[r, pl.ds(k * 16, 16)]
                res = jnp.where(lane == j, jnp.sum(acc), res)
            out_v[pl.ds(q * slab + base, 16)] = res
            return 0

        lax.fori_loop(0, slab // 16, group_body, 0)
        return 0

    lax.fori_loop(0, 8, q_body, 0)
    pltpu.sync_copy(out_v, out_hbm.at[pl.ds(wid * B_PER_W, B_PER_W)])


@jax.jit
def _run(uf2d, if2d, ut_t, it_t, tail_u, tail_i):
    rows_t = jax.ShapeDtypeStruct((BATCH + 16, 128), jnp.float32)
    sweep = functools.partial(
        pl.kernel,
        out_type=(rows_t, rows_t),
        mesh=plsc.VectorSubcoreMesh(**_MESH),
        compiler_params=_PARAMS,
        scratch_types=[
            pltpu.VMEM((BATCH // 128, 128), jnp.int32),
            pltpu.VMEM((BATCH + 16,), jnp.int32),
            pltpu.VMEM((DIM, 2 * U_CHUNK + 128), jnp.float32),
            pltpu.VMEM((48,), jnp.int32),
            pltpu.VMEM((2, 16, 128), jnp.float32),
            pltpu.VMEM((2, 16), jnp.int32),
            pltpu.SemaphoreType.DMA,
            pltpu.SemaphoreType.DMA,
        ],
    )(_sweep_body)
    rows_u, rows_i = sweep(uf2d, if2d, ut_t, it_t, tail_u, tail_i)

    comb = functools.partial(
        pl.kernel,
        out_type=jax.ShapeDtypeStruct((BATCH,), jnp.float32),
        mesh=plsc.VectorSubcoreMesh(**_MESH),
        compiler_params=_PARAMS,
        scratch_types=[
            pltpu.VMEM((B_PER_W // 8, 128), jnp.float32),
            pltpu.VMEM((B_PER_W // 8, 128), jnp.float32),
            pltpu.VMEM((B_PER_W,), jnp.float32),
        ],
    )(_combine_body)
    return comb(rows_u, rows_i)


def kernel(user_feature, item_feature, user_table, item_table):
    uf2d = user_feature.astype(jnp.int32).reshape(BATCH // 128, 128)
    if2d = item_feature.astype(jnp.int32).reshape(BATCH // 128, 128)
    ut_t = user_table.T
    it_t = item_table.T
    tail_u = jnp.pad(user_table[U_ALIGNED:].T, ((0, 0), (0, 128 - U_TAIL)))
    tail_i = jnp.pad(item_table[I_ALIGNED:].T, ((0, 0), (0, 256 - I_TAIL)))
    return _run(uf2d, if2d, ut_t, it_t, tail_u, tail_i)
